# baseline (speedup 1.0000x reference)
import functools

import jax
import jax.numpy as jnp
from jax import lax
from jax.experimental import pallas as pl
from jax.experimental.pallas import tpu as pltpu
from jax.experimental.pallas import tpu_sc as plsc

VOCAB = 100000
N_DIM = 768
BATCH = 4
SEQ = 8192

NUM_CORES = 2
NUM_SUBCORES = 16
NUM_WORKERS = NUM_CORES * NUM_SUBCORES

LANES = 16
VECS_PER_ROW = N_DIM // LANES
S_PER_WORKER = SEQ // NUM_WORKERS
CHUNK = 16
POS_CHUNKS = S_PER_WORKER // CHUNK
LAST = POS_CHUNKS - 1
NWORK = 4
NPOS = 2


def _make_sc_kernel():
  mesh = plsc.VectorSubcoreMesh(
      core_axis_name="c", subcore_axis_name="s", num_cores=NUM_CORES
  )

  @functools.partial(
      pl.kernel,
      out_type=jax.ShapeDtypeStruct((BATCH * SEQ, N_DIM), jnp.float32),
      mesh=mesh,
      scratch_types=[
          pltpu.VMEM((BATCH, S_PER_WORKER), jnp.int32),
          [pltpu.VMEM((CHUNK, N_DIM), jnp.float32)] * NWORK,
          [pltpu.VMEM((CHUNK, N_DIM), jnp.float32)] * NPOS,
          [pltpu.SemaphoreType.DMA] * NWORK,
          [pltpu.SemaphoreType.DMA] * NWORK,
          [pltpu.SemaphoreType.DMA] * NPOS,
      ],
  )
  def sc_kernel(table_hbm, tokens_hbm, pos_hbm, out_hbm,
                idx_v, work, posb, gsem, wsem, psem):
    wid = lax.axis_index("s") * NUM_CORES + lax.axis_index("c")
    s0 = wid * S_PER_WORKER

    for b in range(BATCH):
      pltpu.sync_copy(
          tokens_hbm.at[pl.ds(b * SEQ + s0, S_PER_WORKER)], idx_v.at[b]
      )

    def add_pos(buf, pv):
      @plsc.parallel_loop(0, CHUNK, 1, unroll=4)
      def _(i):
        for j in range(VECS_PER_ROW):
          sl = pl.ds(j * LANES, LANES)
          plsc.addupdate(buf.at[i, sl], pv[i, sl])

    def issue_pos(pc, ps):
      return pltpu.async_copy(
          pos_hbm.at[pl.ds(s0 + pc * CHUNK, CHUNK)], posb[ps], psem[ps]
      )

    def issue_gather(pc, b):
      return pltpu.async_copy(
          table_hbm.at[idx_v.at[b, pl.ds(pc * CHUNK, CHUNK)]], work[b],
          gsem[b],
      )

    def issue_write(pc, b):
      row0 = b * SEQ + s0 + pc * CHUNK
      return pltpu.async_copy(work[b], out_hbm.at[pl.ds(row0, CHUNK)], wsem[b])

    def wait_gather(b):
      pltpu.make_async_copy(
          table_hbm.at[pl.ds(0, CHUNK)], work[b], gsem[b]
      ).wait()

    def wait_write(b):
      pltpu.make_async_copy(
          work[b], out_hbm.at[pl.ds(0, CHUNK)], wsem[b]
      ).wait()

    def chunk_step(pc, ps):
      @pl.when(pc > 0)
      def _():
        wait_write(3)
      issue_gather(pc, 3)

      wait_gather(0)
      pltpu.make_async_copy(
          pos_hbm.at[pl.ds(0, CHUNK)], posb[ps], psem[ps]
      ).wait()

      @pl.when(pc < LAST)
      def _():
        issue_pos(pc + 1, 1 - ps)
      add_pos(work[0], posb[ps])
      issue_write(pc, 0)

      wait_gather(1)
      add_pos(work[1], posb[ps])
      issue_write(pc, 1)

      wait_write(0)

      @pl.when(pc < LAST)
      def _():
        issue_gather(pc + 1, 0)

      wait_gather(2)
      add_pos(work[2], posb[ps])
      issue_write(pc, 2)

      wait_write(1)

      @pl.when(pc < LAST)
      def _():
        issue_gather(pc + 1, 1)

      wait_gather(3)
      add_pos(work[3], posb[ps])
      issue_write(pc, 3)

      wait_write(2)

      @pl.when(pc < LAST)
      def _():
        issue_gather(pc + 1, 2)

    issue_pos(0, 0)
    for b in range(3):
      issue_gather(0, b)

    def body(k, _):
      chunk_step(2 * k, 0)
      chunk_step(2 * k + 1, 1)
      return 0

    lax.fori_loop(0, POS_CHUNKS // 2, body, 0)
    wait_write(3)

  return sc_kernel


_sc_kernel = _make_sc_kernel()


@jax.jit
def kernel(tokens, token_table, pos_table):
  tokens_flat = tokens.reshape(-1).astype(jnp.int32)
  out = _sc_kernel(token_table, tokens_flat, pos_table)
  return out.reshape(BATCH, SEQ, N_DIM)


# speedup vs baseline: 1.4280x; 1.4280x over previous
import functools

import jax
import jax.numpy as jnp
from jax import lax
from jax.experimental import pallas as pl
from jax.experimental.pallas import tpu as pltpu
from jax.experimental.pallas import tpu_sc as plsc

VOCAB = 100000
N_DIM = 768
BATCH = 4
SEQ = 8192

NUM_CORES = 2
NUM_SUBCORES = 16
NUM_WORKERS = NUM_CORES * NUM_SUBCORES

LANES = 16
VECS_PER_ROW = N_DIM // LANES
S_PER_WORKER = SEQ // NUM_WORKERS
CHUNK = 16
POS_CHUNKS = S_PER_WORKER // CHUNK
LAST = POS_CHUNKS - 1
NWORK = 4
NPOS = 2


def _make_sc_kernel():
  mesh = plsc.VectorSubcoreMesh(
      core_axis_name="c", subcore_axis_name="s", num_cores=NUM_CORES
  )

  @functools.partial(
      pl.kernel,
      out_type=jax.ShapeDtypeStruct((BATCH * SEQ, N_DIM), jnp.float32),
      mesh=mesh,
      scratch_types=[
          pltpu.VMEM((BATCH, S_PER_WORKER), jnp.int32),
          [pltpu.VMEM((CHUNK, N_DIM), jnp.float32)] * NWORK,
          [pltpu.VMEM((CHUNK, N_DIM), jnp.float32)] * NPOS,
          [pltpu.SemaphoreType.DMA] * NWORK,
          [pltpu.SemaphoreType.DMA] * NWORK,
          [pltpu.SemaphoreType.DMA] * NPOS,
      ],
  )
  def sc_kernel(table_hbm, tokens_hbm, pos_hbm, out_hbm,
                idx_v, work, posb, gsem, wsem, psem):
    wid = lax.axis_index("s") * NUM_CORES + lax.axis_index("c")
    s0 = wid * S_PER_WORKER

    for b in range(BATCH):
      pltpu.sync_copy(
          tokens_hbm.at[pl.ds(b * SEQ + s0, S_PER_WORKER)], idx_v.at[b]
      )

    def add_pos(buf, pv):
      @plsc.parallel_loop(0, CHUNK, 1, unroll=2)
      def _(i):
        for j in range(VECS_PER_ROW):
          sl = pl.ds(j * LANES, LANES)
          plsc.addupdate(buf.at[i, sl], pv[i, sl])

    def issue_pos(pc, ps):
      return pltpu.async_copy(
          pos_hbm.at[pl.ds(s0 + pc * CHUNK, CHUNK)], posb[ps], psem[ps]
      )

    def issue_gather(pc, b):
      return pltpu.async_copy(
          table_hbm.at[idx_v.at[b, pl.ds(pc * CHUNK, CHUNK)]], work[b],
          gsem[b],
      )

    def issue_write(pc, b):
      row0 = b * SEQ + s0 + pc * CHUNK
      return pltpu.async_copy(work[b], out_hbm.at[pl.ds(row0, CHUNK)], wsem[b])

    def wait_gather(b):
      pltpu.make_async_copy(
          table_hbm.at[pl.ds(0, CHUNK)], work[b], gsem[b]
      ).wait()

    def wait_write(b):
      pltpu.make_async_copy(
          work[b], out_hbm.at[pl.ds(0, CHUNK)], wsem[b]
      ).wait()

    def chunk_step(pc, ps):
      @pl.when(pc > 0)
      def _():
        wait_write(3)
      issue_gather(pc, 3)

      wait_gather(0)
      pltpu.make_async_copy(
          pos_hbm.at[pl.ds(0, CHUNK)], posb[ps], psem[ps]
      ).wait()

      @pl.when(pc < LAST)
      def _():
        issue_pos(pc + 1, 1 - ps)
      add_pos(work[0], posb[ps])
      issue_write(pc, 0)

      wait_gather(1)
      add_pos(work[1], posb[ps])
      issue_write(pc, 1)

      wait_write(0)

      @pl.when(pc < LAST)
      def _():
        issue_gather(pc + 1, 0)

      wait_gather(2)
      add_pos(work[2], posb[ps])
      issue_write(pc, 2)

      wait_write(1)

      @pl.when(pc < LAST)
      def _():
        issue_gather(pc + 1, 1)

      wait_gather(3)
      add_pos(work[3], posb[ps])
      issue_write(pc, 3)

      wait_write(2)

      @pl.when(pc < LAST)
      def _():
        issue_gather(pc + 1, 2)

    issue_pos(0, 0)
    for b in range(3):
      issue_gather(0, b)

    def body(k, _):
      chunk_step(2 * k, 0)
      chunk_step(2 * k + 1, 1)
      return 0

    lax.fori_loop(0, POS_CHUNKS // 2, body, 0)
    wait_write(3)

  return sc_kernel


_sc_kernel = _make_sc_kernel()


@jax.jit
def kernel(tokens, token_table, pos_table):
  tokens_flat = tokens.reshape(-1).astype(jnp.int32)
  out = _sc_kernel(token_table, tokens_flat, pos_table)
  return out.reshape(BATCH, SEQ, N_DIM)


# speedup vs baseline: 1.4509x; 1.0161x over previous
import functools

import jax
import jax.numpy as jnp
from jax import lax
from jax.experimental import pallas as pl
from jax.experimental.pallas import tpu as pltpu
from jax.experimental.pallas import tpu_sc as plsc

VOCAB = 100000
N_DIM = 768
BATCH = 4
SEQ = 8192

NUM_CORES = 2
NUM_SUBCORES = 16
NUM_WORKERS = NUM_CORES * NUM_SUBCORES

LANES = 16
VECS_PER_ROW = N_DIM // LANES
S_PER_WORKER = SEQ // NUM_WORKERS
CHUNK = 16
POS_CHUNKS = S_PER_WORKER // CHUNK
LAST = POS_CHUNKS - 1
NWORK = 8
NPOS = 2


def _make_sc_kernel():
  mesh = plsc.VectorSubcoreMesh(
      core_axis_name="c", subcore_axis_name="s", num_cores=NUM_CORES
  )

  @functools.partial(
      pl.kernel,
      out_type=jax.ShapeDtypeStruct((BATCH * SEQ, N_DIM), jnp.float32),
      mesh=mesh,
      scratch_types=[
          pltpu.VMEM((BATCH, S_PER_WORKER), jnp.int32),
          [pltpu.VMEM((CHUNK, N_DIM), jnp.float32)] * NWORK,
          [pltpu.VMEM((CHUNK, N_DIM), jnp.float32)] * NPOS,
          [pltpu.SemaphoreType.DMA] * NWORK,
          [pltpu.SemaphoreType.DMA] * NWORK,
          [pltpu.SemaphoreType.DMA] * NPOS,
      ],
  )
  def sc_kernel(table_hbm, tokens_hbm, pos_hbm, out_hbm,
                idx_v, work, posb, gsem, wsem, psem):
    wid = lax.axis_index("s") * NUM_CORES + lax.axis_index("c")
    s0 = wid * S_PER_WORKER

    for b in range(BATCH):
      pltpu.sync_copy(
          tokens_hbm.at[pl.ds(b * SEQ + s0, S_PER_WORKER)], idx_v.at[b]
      )

    def add_pos(buf, pv):
      @plsc.parallel_loop(0, CHUNK, 1, unroll=2)
      def _(i):
        for j in range(VECS_PER_ROW):
          sl = pl.ds(j * LANES, LANES)
          plsc.addupdate(buf.at[i, sl], pv[i, sl])

    def issue_pos(pc, ps):
      return pltpu.async_copy(
          pos_hbm.at[pl.ds(s0 + pc * CHUNK, CHUNK)], posb[ps], psem[ps]
      )

    def issue_gather(pc, slot, b):
      return pltpu.async_copy(
          table_hbm.at[idx_v.at[b, pl.ds(pc * CHUNK, CHUNK)]], work[slot],
          gsem[slot],
      )

    def issue_write(pc, slot, b):
      row0 = b * SEQ + s0 + pc * CHUNK
      return pltpu.async_copy(
          work[slot], out_hbm.at[pl.ds(row0, CHUNK)], wsem[slot]
      )

    def wait_gather(slot):
      pltpu.make_async_copy(
          table_hbm.at[pl.ds(0, CHUNK)], work[slot], gsem[slot]
      ).wait()

    def wait_write(slot):
      pltpu.make_async_copy(
          work[slot], out_hbm.at[pl.ds(0, CHUNK)], wsem[slot]
      ).wait()

    def chunk_step(pc, bank, ps):
      other = 1 - bank
      for b in range(BATCH):
        slot = bank * BATCH + b
        oslot = other * BATCH + b
        wait_gather(slot)
        if b == 0:
          pltpu.make_async_copy(
              pos_hbm.at[pl.ds(0, CHUNK)], posb[ps], psem[ps]
          ).wait()

          @pl.when(pc < LAST)
          def _():
            issue_pos(pc + 1, 1 - ps)
        add_pos(work[slot], posb[ps])
        issue_write(pc, slot, b)

        @pl.when(pc > 0)
        def _():
          wait_write(oslot)

        @pl.when(pc < LAST)
        def _():
          issue_gather(pc + 1, oslot, b)

    issue_pos(0, 0)
    for b in range(BATCH):
      issue_gather(0, b, b)

    def body(k, _):
      chunk_step(2 * k, 0, 0)
      chunk_step(2 * k + 1, 1, 1)
      return 0

    lax.fori_loop(0, POS_CHUNKS // 2, body, 0)
    for b in range(BATCH):
      wait_write(BATCH + b)

  return sc_kernel


_sc_kernel = _make_sc_kernel()


@jax.jit
def kernel(tokens, token_table, pos_table):
  tokens_flat = tokens.reshape(-1).astype(jnp.int32)
  out = _sc_kernel(token_table, tokens_flat, pos_table)
  return out.reshape(BATCH, SEQ, N_DIM)


# speedup vs baseline: 1.4739x; 1.0158x over previous
import functools

import jax
import jax.numpy as jnp
from jax import lax
from jax.experimental import pallas as pl
from jax.experimental.pallas import tpu as pltpu
from jax.experimental.pallas import tpu_sc as plsc

VOCAB = 100000
N_DIM = 768
BATCH = 4
SEQ = 8192

NUM_CORES = 2
NUM_SUBCORES = 16
NUM_WORKERS = NUM_CORES * NUM_SUBCORES

LANES = 16
VECS_PER_ROW = N_DIM // LANES
S_PER_WORKER = SEQ // NUM_WORKERS
CHUNK = 16
POS_CHUNKS = S_PER_WORKER // CHUNK
LAST = POS_CHUNKS - 1
NWORK = 8
NPOS = 2


def _make_sc_kernel():
  mesh = plsc.VectorSubcoreMesh(
      core_axis_name="c", subcore_axis_name="s", num_cores=NUM_CORES
  )

  @functools.partial(
      pl.kernel,
      out_type=jax.ShapeDtypeStruct((BATCH * SEQ, N_DIM), jnp.float32),
      mesh=mesh,
      scratch_types=[
          pltpu.VMEM((BATCH, S_PER_WORKER), jnp.int32),
          [pltpu.VMEM((CHUNK, N_DIM), jnp.float32)] * NWORK,
          [pltpu.VMEM((CHUNK, N_DIM), jnp.float32)] * NPOS,
          [pltpu.SemaphoreType.DMA] * NWORK,
          [pltpu.SemaphoreType.DMA] * NWORK,
          [pltpu.SemaphoreType.DMA] * NPOS,
      ],
  )
  def sc_kernel(table_hbm, tokens_hbm, pos_hbm, out_hbm,
                idx_v, work, posb, gsem, wsem, psem):
    wid = lax.axis_index("s") * NUM_CORES + lax.axis_index("c")
    s0 = wid * S_PER_WORKER

    pltpu.sync_copy(tokens_hbm.at[:, pl.ds(s0, S_PER_WORKER)], idx_v)

    def add_pos(buf, pv):
      @plsc.parallel_loop(0, CHUNK, 1, unroll=2)
      def _(i):
        for j in range(VECS_PER_ROW):
          sl = pl.ds(j * LANES, LANES)
          plsc.addupdate(buf.at[i, sl], pv[i, sl])

    def issue_pos(pc, ps):
      return pltpu.async_copy(
          pos_hbm.at[pl.ds(s0 + pc * CHUNK, CHUNK)], posb[ps], psem[ps]
      )

    def issue_gather(pc, slot, b):
      return pltpu.async_copy(
          table_hbm.at[idx_v.at[b, pl.ds(pc * CHUNK, CHUNK)]], work[slot],
          gsem[slot],
      )

    def issue_write(pc, slot, b):
      row0 = b * SEQ + s0 + pc * CHUNK
      return pltpu.async_copy(
          work[slot], out_hbm.at[pl.ds(row0, CHUNK)], wsem[slot]
      )

    def wait_gather(slot):
      pltpu.make_async_copy(
          table_hbm.at[pl.ds(0, CHUNK)], work[slot], gsem[slot]
      ).wait()

    def wait_write(slot):
      pltpu.make_async_copy(
          work[slot], out_hbm.at[pl.ds(0, CHUNK)], wsem[slot]
      ).wait()

    def chunk_step(pc, bank, ps):
      other = 1 - bank
      for b in range(BATCH):
        slot = bank * BATCH + b
        oslot = other * BATCH + b
        wait_gather(slot)
        if b == 0:
          pltpu.make_async_copy(
              pos_hbm.at[pl.ds(0, CHUNK)], posb[ps], psem[ps]
          ).wait()

          @pl.when(pc < LAST)
          def _():
            issue_pos(pc + 1, 1 - ps)
        @pl.when(pc > 0)
        def _():
          wait_write(oslot)

        @pl.when(pc < LAST)
        def _():
          issue_gather(pc + 1, oslot, b)

        add_pos(work[slot], posb[ps])
        issue_write(pc, slot, b)

    issue_pos(0, 0)
    for b in range(BATCH):
      issue_gather(0, b, b)

    def body(k, _):
      chunk_step(2 * k, 0, 0)
      chunk_step(2 * k + 1, 1, 1)
      return 0

    lax.fori_loop(0, POS_CHUNKS // 2, body, 0)
    for b in range(BATCH):
      wait_write(BATCH + b)

  return sc_kernel


_sc_kernel = _make_sc_kernel()


@jax.jit
def kernel(tokens, token_table, pos_table):
  out = _sc_kernel(token_table, tokens.astype(jnp.int32), pos_table)
  return out.reshape(BATCH, SEQ, N_DIM)
